# baseline (speedup 1.0000x reference)
import functools

import jax
import jax.numpy as jnp
from jax import lax
from jax.experimental import pallas as pl
from jax.experimental.pallas import tpu as pltpu
from jax.experimental.pallas import tpu_sc as plsc
from jax.experimental import layout as jxl

RADIUS2 = 4.0
NSAMPLE = 32
NCAND = 180
NCP = 192
M = 16384
CFEAT = 16
RV_H, RV_W, PPP = 64, 2048, 4

NCORES, NSUBC = 2, 16
NW = NCORES * NSUBC
QPW = M // NW
G = 16
NGRP = QPW // G
CH = 128
NCH_CAND = (G * NCP) // CH
NCH_SEL = (G * NSAMPLE) // CH
OW = (3 + CFEAT) * NSAMPLE


def _splat(x, dtype=jnp.int32):
    return jnp.full((16,), x, dtype=dtype)


def _vgather(v, idx):
    return v.at[idx].get(mode="promise_in_bounds")


def _elem(buf, pos):
    return plsc.load_gather(buf, [_splat(pos >> 7), _splat(pos & 127)])


def _sc_body(q_h, c_h, rvf_h, xyzp_h, feat_h, out_h,
             qbuf, cbuf, eidx, cand, cxyz, sel, gfeat, gxyz,
             cntv, outb, sem):
    wid = lax.axis_index("s") * NCORES + lax.axis_index("c")
    qbase = wid * QPW
    qrow = wid * (QPW * 3 // CH)
    iota = jnp.arange(16, dtype=jnp.int32)

    pltpu.sync_copy(q_h.at[pl.ds(qrow, QPW * 3 // CH)], qbuf)
    pltpu.sync_copy(c_h.at[pl.ds(qrow, QPW * 3 // CH)], cbuf)

    def group_body(g, carry):
        def build_body(i, bc):
            lq = g * G + i
            rsp = _elem(cbuf, lq * 3 + 1) & jnp.int32(RV_H - 1)
            csp = _elem(cbuf, lq * 3 + 2) & jnp.int32(RV_W - 1)
            cells = []
            for jj in range(3):
                u = iota + 16 * jj
                oh = u // 9 - 2
                ow = 2 * (u % 9) - 8
                rr = jnp.clip(rsp + oh, 0, RV_H - 1)
                cc = (csp + ow) & jnp.int32(RV_W - 1)
                cells.append(rr * RV_W + cc)
            for jj2 in range(12):
                lidx = iota // 4 + 4 * (jj2 % 4)
                cv = _vgather(cells[jj2 // 4], lidx)
                ev = cv * PPP + (iota & 3)
                p = _splat(i * NCP + 16 * jj2) + iota
                plsc.store_scatter(eidx, [p >> 7, p & 127], ev)
            return bc

        lax.fori_loop(0, G, build_body, 0)

        cps = []
        for j in range(NCH_CAND):
            cp = pltpu.make_async_copy(rvf_h.at[eidx.at[j]], cand.at[j], sem)
            cp.start()
            cps.append(cp)
        for cp in cps:
            cp.wait()

        cps = []
        for j in range(NCH_CAND):
            cp = pltpu.make_async_copy(xyzp_h.at[cand.at[j]], cxyz.at[j], sem)
            cp.start()
            cps.append(cp)
        for cp in cps:
            cp.wait()

        def select_body(i, bc):
            lq = g * G + i
            xq = _elem(qbuf, lq * 3)
            yq = _elem(qbuf, lq * 3 + 1)
            zq = _elem(qbuf, lq * 3 + 2)
            z16 = _splat(0)
            cnt = jnp.int32(0)
            for jj in range(12):
                p = _splat(i * NCP + 16 * jj) + iota
                pr, pc = p >> 7, p & 127
                cd = plsc.load_gather(cand, [pr, pc])
                x = plsc.load_gather(cxyz, [pr, pc, z16])
                y = plsc.load_gather(cxyz, [pr, pc, z16 + 1])
                z = plsc.load_gather(cxyz, [pr, pc, z16 + 2])
                dx, dy, dz = x - xq, y - yq, z - zq
                d2 = dx * dx + dy * dy + dz * dz
                val = d2 < RADIUS2
                if jj == 11:
                    val = val & (iota < (NCAND - 16 * 11))
                vi = val.astype(jnp.int32)
                pref = plsc.cumsum(vi)
                rank = cnt + pref - 1
                m = val & (rank < NSAMPLE)
                sp = _splat(i * NSAMPLE) + rank
                plsc.store_scatter(sel, [sp >> 7, sp & 127], cd, mask=m)
                cnt = cnt + jnp.sum(vi)
            sp0 = i * NSAMPLE
            fsv = plsc.load_gather(
                sel, [_splat(0) + (sp0 >> 7), _splat(0) + (sp0 & 127)])
            for h in range(2):
                k = iota + 16 * h
                spk = sp0 + k
                cur = plsc.load_gather(sel, [spk >> 7, spk & 127])
                new = jnp.where(k < cnt, cur, fsv)
                new = jnp.where(cnt > 0, new, 0)
                plsc.store_scatter(sel, [spk >> 7, spk & 127], new)
            plsc.store_scatter(cntv, [_splat(0), _splat(0) + i],
                               _splat(0) + cnt, mask=iota == 0)
            return bc

        lax.fori_loop(0, G, select_body, 0)

        cps = []
        for j in range(NCH_SEL):
            cp = pltpu.make_async_copy(feat_h.at[sel.at[j]], gfeat.at[j], sem)
            cp.start()
            cps.append(cp)
            cp = pltpu.make_async_copy(xyzp_h.at[sel.at[j]], gxyz.at[j], sem)
            cp.start()
            cps.append(cp)
        for cp in cps:
            cp.wait()

        def out_body(i, bc):
            lq = g * G + i
            xq = _elem(qbuf, lq * 3)
            yq = _elem(qbuf, lq * 3 + 1)
            zq = _elem(qbuf, lq * 3 + 2)
            cz = plsc.load_gather(cntv, [_splat(0), _splat(0) + i]) > 0
            qs = (xq, yq, zq)
            for c in range(3 + CFEAT):
                for h in range(2):
                    sp = _splat(i * NSAMPLE + 16 * h) + iota
                    sr, sc = sp >> 7, sp & 127
                    if c < 3:
                        v = plsc.load_gather(gxyz, [sr, sc, _splat(c)]) - qs[c]
                    else:
                        v = plsc.load_gather(gfeat, [sr, sc, _splat(c - 3)])
                    v = jnp.where(cz, v, 0.0)
                    k = _splat(16 * h) + iota
                    plsc.store_scatter(
                        outb, [_splat(0) + i, _splat(c), k], v)
            return bc

        lax.fori_loop(0, G, out_body, 0)

        pltpu.sync_copy(outb, out_h.at[pl.ds(qbase + g * G, G)])
        return carry

    lax.fori_loop(0, NGRP, group_body, 0)


@jax.jit
def kernel(xyz, features, query_rv_xyz, query_rv_coords, rv_map):
    xyzp = jnp.concatenate(
        [xyz, jnp.zeros((xyz.shape[0], 5), jnp.float32)], axis=1)
    rvf = rv_map.reshape(-1)
    qv = query_rv_xyz.reshape(M * 3 // CH, CH)
    cv = query_rv_coords.reshape(M * 3 // CH, CH)

    mesh = plsc.VectorSubcoreMesh(core_axis_name="c", subcore_axis_name="s",
                                  num_cores=NCORES, num_subcores=NSUBC)
    run = pl.kernel(
        _sc_body,
        out_type=jax.ShapeDtypeStruct((M, 3 + CFEAT, NSAMPLE),
                                      jnp.float32),
        mesh=mesh,
        compiler_params=pltpu.CompilerParams(use_tc_tiling_on_sc=False,
                                             needs_layout_passes=False),
        scratch_types=[
            pltpu.VMEM((QPW * 3 // CH, CH), jnp.float32),
            pltpu.VMEM((QPW * 3 // CH, CH), jnp.int32),
            pltpu.VMEM((NCH_CAND, CH), jnp.int32),
            pltpu.VMEM((NCH_CAND, CH), jnp.int32),
            pltpu.VMEM((NCH_CAND, CH, 8), jnp.float32),
            pltpu.VMEM((NCH_SEL, CH), jnp.int32),
            pltpu.VMEM((NCH_SEL, CH, CFEAT), jnp.float32),
            pltpu.VMEM((NCH_SEL, CH, 8), jnp.float32),
            pltpu.VMEM((1, CH), jnp.int32),
            pltpu.VMEM((G, 3 + CFEAT, NSAMPLE), jnp.float32),
            pltpu.SemaphoreType.DMA,
        ],
    )
    out = run(qv, cv, rvf, xyzp, features)
    return jxl.with_layout_constraint(
        out, jxl.Layout(major_to_minor=(0, 1, 2), tiling=()))


# speedup vs baseline: 1.0020x; 1.0020x over previous
import functools

import jax
import jax.numpy as jnp
from jax import lax
from jax.experimental import pallas as pl
from jax.experimental.pallas import tpu as pltpu
from jax.experimental.pallas import tpu_sc as plsc
from jax.experimental import layout as jxl

RADIUS2 = 4.0
NSAMPLE = 32
NCAND = 180
NCP = 192
M = 16384
CFEAT = 16
RV_H, RV_W, PPP = 64, 2048, 4

NCORES, NSUBC = 2, 16
NW = NCORES * NSUBC
QPW = M // NW
G = 16
NGRP = QPW // G
CH = 128
NCH_CAND = (G * NCP) // CH
NCH_SEL = (G * NSAMPLE) // CH
OW = (3 + CFEAT) * NSAMPLE


def _splat(x, dtype=jnp.int32):
    return jnp.full((16,), x, dtype=dtype)


def _vgather(v, idx):
    return v.at[idx].get(mode="promise_in_bounds")


def _elem(buf, pos):
    return plsc.load_gather(buf, [_splat(pos >> 7), _splat(pos & 127)])


def _sc_body(q_h, c_h, rvf_h, xyzp_h, feat_h, out_h,
             qbuf, cbuf, eidx, cand, cxyz, sel, gfeat, gxyz,
             cntv, outb, sem):
    wid = lax.axis_index("s") * NCORES + lax.axis_index("c")
    qbase = wid * QPW
    qrow = wid * (QPW * 3 // CH)
    iota = jnp.arange(16, dtype=jnp.int32)

    pltpu.sync_copy(q_h.at[pl.ds(qrow, QPW * 3 // CH)], qbuf)
    pltpu.sync_copy(c_h.at[pl.ds(qrow, QPW * 3 // CH)], cbuf)

    def group_body(g, carry):
        def build_body(i, bc):
            lq = g * G + i
            rsp = _elem(cbuf, lq * 3 + 1) & jnp.int32(RV_H - 1)
            csp = _elem(cbuf, lq * 3 + 2) & jnp.int32(RV_W - 1)
            cells = []
            for jj in range(3):
                u = iota + 16 * jj
                oh = u // 9 - 2
                ow = 2 * (u % 9) - 8
                rr = jnp.clip(rsp + oh, 0, RV_H - 1)
                cc = (csp + ow) & jnp.int32(RV_W - 1)
                cells.append(rr * RV_W + cc)
            for jj2 in range(12):
                lidx = iota // 4 + 4 * (jj2 % 4)
                cv = _vgather(cells[jj2 // 4], lidx)
                ev = cv * PPP + (iota & 3)
                p = _splat(i * NCP + 16 * jj2) + iota
                plsc.store_scatter(eidx, [p >> 7, p & 127], ev)
            return bc

        lax.fori_loop(0, G, build_body, 0)

        cps = []
        for j in range(NCH_CAND):
            cp = pltpu.make_async_copy(rvf_h.at[eidx.at[j]], cand.at[j], sem)
            cp.start()
            cps.append(cp)
        for cp in cps:
            cp.wait()

        cps = []
        for j in range(NCH_CAND):
            cp = pltpu.make_async_copy(xyzp_h.at[cand.at[j]], cxyz.at[j], sem)
            cp.start()
            cps.append(cp)
        for cp in cps:
            cp.wait()

        def select_body(i, bc):
            lq = g * G + i
            xq = _elem(qbuf, lq * 3)
            yq = _elem(qbuf, lq * 3 + 1)
            zq = _elem(qbuf, lq * 3 + 2)
            z16 = _splat(0)
            cnt = jnp.int32(0)
            for jj in range(12):
                p = _splat(i * NCP + 16 * jj) + iota
                pr, pc = p >> 7, p & 127
                cd = plsc.load_gather(cand, [pr, pc])
                x = plsc.load_gather(cxyz, [pr, pc, z16])
                y = plsc.load_gather(cxyz, [pr, pc, z16 + 1])
                z = plsc.load_gather(cxyz, [pr, pc, z16 + 2])
                dx, dy, dz = x - xq, y - yq, z - zq
                d2 = dx * dx + dy * dy + dz * dz
                val = d2 < RADIUS2
                if jj == 11:
                    val = val & (iota < (NCAND - 16 * 11))
                vi = val.astype(jnp.int32)
                pref = plsc.cumsum(vi)
                rank = cnt + pref - 1
                m = val & (rank < NSAMPLE)
                sp = _splat(i * NSAMPLE) + rank
                plsc.store_scatter(sel, [sp >> 7, sp & 127], cd, mask=m)
                cnt = cnt + jnp.sum(vi)
            sp0 = i * NSAMPLE
            fsv = plsc.load_gather(
                sel, [_splat(0) + (sp0 >> 7), _splat(0) + (sp0 & 127)])
            for h in range(2):
                k = iota + 16 * h
                spk = sp0 + k
                cur = plsc.load_gather(sel, [spk >> 7, spk & 127])
                new = jnp.where(k < cnt, cur, fsv)
                new = jnp.where(cnt > 0, new, 0)
                plsc.store_scatter(sel, [spk >> 7, spk & 127], new)
            plsc.store_scatter(cntv, [_splat(0), _splat(0) + i],
                               _splat(0) + cnt, mask=iota == 0)
            return bc

        lax.fori_loop(0, G, select_body, 0)

        cps = []
        for j in range(NCH_SEL):
            cp = pltpu.make_async_copy(feat_h.at[sel.at[j]], gfeat.at[j], sem)
            cp.start()
            cps.append(cp)
            cp = pltpu.make_async_copy(xyzp_h.at[sel.at[j]], gxyz.at[j], sem)
            cp.start()
            cps.append(cp)
        for cp in cps:
            cp.wait()

        def out_body(i, bc):
            lq = g * G + i
            xq = _elem(qbuf, lq * 3)
            yq = _elem(qbuf, lq * 3 + 1)
            zq = _elem(qbuf, lq * 3 + 2)
            cz = plsc.load_gather(cntv, [_splat(0), _splat(0) + i]) > 0
            qs = (xq, yq, zq)
            for c in range(3 + CFEAT):
                for h in range(2):
                    sp = _splat(i * NSAMPLE + 16 * h) + iota
                    sr, sc = sp >> 7, sp & 127
                    if c < 3:
                        v = plsc.load_gather(gxyz, [sr, sc, _splat(c)]) - qs[c]
                    else:
                        v = plsc.load_gather(gfeat, [sr, sc, _splat(c - 3)])
                    v = jnp.where(cz, v, 0.0)
                    k = _splat(16 * h) + iota
                    plsc.store_scatter(
                        outb, [_splat(0) + i, _splat(c), k], v)
            return bc

        lax.fori_loop(0, G, out_body, 0)

        pltpu.sync_copy(outb, out_h.at[pl.ds(qbase + g * G, G)])
        return carry

    lax.fori_loop(0, NGRP, group_body, 0)


def _impl(xyz, features, query_rv_xyz, query_rv_coords, rv_map):
    xyzp = jnp.concatenate(
        [xyz, jnp.zeros((xyz.shape[0], 5), jnp.float32)], axis=1)
    rvf = rv_map.reshape(-1)
    qv = query_rv_xyz.reshape(M * 3 // CH, CH)
    cv = query_rv_coords.reshape(M * 3 // CH, CH)

    mesh = plsc.VectorSubcoreMesh(core_axis_name="c", subcore_axis_name="s",
                                  num_cores=NCORES, num_subcores=NSUBC)
    run = pl.kernel(
        _sc_body,
        out_type=jax.ShapeDtypeStruct((M, 3 + CFEAT, NSAMPLE),
                                      jnp.float32),
        mesh=mesh,
        compiler_params=pltpu.CompilerParams(use_tc_tiling_on_sc=False,
                                             needs_layout_passes=False),
        scratch_types=[
            pltpu.VMEM((QPW * 3 // CH, CH), jnp.float32),
            pltpu.VMEM((QPW * 3 // CH, CH), jnp.int32),
            pltpu.VMEM((NCH_CAND, CH), jnp.int32),
            pltpu.VMEM((NCH_CAND, CH), jnp.int32),
            pltpu.VMEM((NCH_CAND, CH, 8), jnp.float32),
            pltpu.VMEM((NCH_SEL, CH), jnp.int32),
            pltpu.VMEM((NCH_SEL, CH, CFEAT), jnp.float32),
            pltpu.VMEM((NCH_SEL, CH, 8), jnp.float32),
            pltpu.VMEM((1, CH), jnp.int32),
            pltpu.VMEM((G, 3 + CFEAT, NSAMPLE), jnp.float32),
            pltpu.SemaphoreType.DMA,
        ],
    )
    return run(qv, cv, rvf, xyzp, features)


_impl.__name__ = "kernel"
_JIT = None


def kernel(xyz, features, query_rv_xyz, query_rv_coords, rv_map):
    global _JIT
    if _JIT is None:
        fmt = jxl.Format(
            jxl.Layout(major_to_minor=(0, 1, 2), tiling=()),
            jax.sharding.SingleDeviceSharding(jax.devices()[0]))
        _JIT = jax.jit(_impl, out_shardings=fmt)
    return _JIT(xyz, features, query_rv_xyz, query_rv_coords, rv_map)


# speedup vs baseline: 1.1220x; 1.1198x over previous
import functools

import jax
import jax.numpy as jnp
from jax import lax
from jax.experimental import pallas as pl
from jax.experimental.pallas import tpu as pltpu
from jax.experimental.pallas import tpu_sc as plsc

RADIUS2 = 4.0
NSAMPLE = 32
NCAND = 180
NCP = 192
M = 16384
CFEAT = 16
RV_H, RV_W, PPP = 64, 2048, 4

NCORES, NSUBC = 2, 16
NW = NCORES * NSUBC
QPW = M // NW
G = 16
NGRP = QPW // G
CH = 128
NCH_CAND = (G * NCP) // CH
NCH_SEL = (G * NSAMPLE) // CH
OW = (3 + CFEAT) * NSAMPLE


def _splat(x, dtype=jnp.int32):
    return jnp.full((16,), x, dtype=dtype)


def _vgather(v, idx):
    return v.at[idx].get(mode="promise_in_bounds")


def _elem(buf, pos):
    return plsc.load_gather(buf, [_splat(pos >> 7), _splat(pos & 127)])


def _sc_body(q_h, c_h, rvf_h, xyzp_h, feat_h, out_h,
             qbuf, cbuf, eidx0, eidx1, cand0, cand1, cxyz0, cxyz1,
             sel, gfeat, gxyz, cntv, outb, semb, semc, seme):
    wid = lax.axis_index("s") * NCORES + lax.axis_index("c")
    qbase = wid * QPW
    qrow = wid * (QPW * 3 // CH)
    iota = jnp.arange(16, dtype=jnp.int32)

    pltpu.sync_copy(q_h.at[pl.ds(qrow, QPW * 3 // CH)], qbuf)
    pltpu.sync_copy(c_h.at[pl.ds(qrow, QPW * 3 // CH)], cbuf)

    eidxs = (eidx0, eidx1)
    cands = (cand0, cand1)
    cxyzs = (cxyz0, cxyz1)

    def build_group(g, eidx):
        def build_body(i, bc):
            lq = g * G + i
            rsp = _elem(cbuf, lq * 3 + 1) & jnp.int32(RV_H - 1)
            csp = _elem(cbuf, lq * 3 + 2) & jnp.int32(RV_W - 1)
            cells = []
            for jj in range(3):
                u = iota + 16 * jj
                oh = u // 9 - 2
                ow = 2 * (u % 9) - 8
                rr = jnp.clip(rsp + oh, 0, RV_H - 1)
                cc = (csp + ow) & jnp.int32(RV_W - 1)
                cells.append(rr * RV_W + cc)
            for jj2 in range(12):
                lidx = iota // 4 + 4 * (jj2 % 4)
                cv = _vgather(cells[jj2 // 4], lidx)
                ev = cv * PPP + (iota & 3)
                p = _splat(i * NCP + 16 * jj2) + iota
                plsc.store_scatter(eidx, [p >> 7, p & 127], ev)
            return bc

        lax.fori_loop(0, G, build_body, 0)

    def fire_rv(eidx, cand):
        for j in range(NCH_CAND):
            pltpu.make_async_copy(rvf_h.at[eidx.at[j]], cand.at[j],
                                  semb).start()

    def drain_rv(eidx, cand):
        for j in range(NCH_CAND):
            pltpu.make_async_copy(rvf_h.at[eidx.at[j]], cand.at[j],
                                  semb).wait()

    build_group(0, eidx0)
    fire_rv(eidx0, cand0)

    def pair_body(t, carry):
        for par in range(2):
            g = 2 * t + par
            eidx, cand, cxyz = eidxs[par], cands[par], cxyzs[par]
            neidx, ncand = eidxs[1 - par], cands[1 - par]

            drain_rv(eidx, cand)
            ccps = []
            for j in range(NCH_CAND):
                cp = pltpu.make_async_copy(xyzp_h.at[cand.at[j]],
                                           cxyz.at[j], semc)
                cp.start()
                ccps.append(cp)

            @pl.when(g < NGRP - 1)
            def _():
                build_group(g + 1, neidx)

            for cp in ccps:
                cp.wait()

            @pl.when(g < NGRP - 1)
            def _():
                fire_rv(neidx, ncand)

            def select_body(i, bc):
                lq = g * G + i
                xq = _elem(qbuf, lq * 3)
                yq = _elem(qbuf, lq * 3 + 1)
                zq = _elem(qbuf, lq * 3 + 2)
                z16 = _splat(0)
                cnt = jnp.int32(0)
                for jj in range(12):
                    p = _splat(i * NCP + 16 * jj) + iota
                    pr, pc = p >> 7, p & 127
                    cd = plsc.load_gather(cand, [pr, pc])
                    x = plsc.load_gather(cxyz, [pr, pc, z16])
                    y = plsc.load_gather(cxyz, [pr, pc, z16 + 1])
                    z = plsc.load_gather(cxyz, [pr, pc, z16 + 2])
                    dx, dy, dz = x - xq, y - yq, z - zq
                    d2 = dx * dx + dy * dy + dz * dz
                    val = d2 < RADIUS2
                    if jj == 11:
                        val = val & (iota < (NCAND - 16 * 11))
                    vi = val.astype(jnp.int32)
                    pref = plsc.cumsum(vi)
                    rank = cnt + pref - 1
                    m = val & (rank < NSAMPLE)
                    sp = _splat(i * NSAMPLE) + rank
                    plsc.store_scatter(sel, [sp >> 7, sp & 127], cd, mask=m)
                    cnt = cnt + jnp.sum(vi)
                sp0 = i * NSAMPLE
                fsv = plsc.load_gather(
                    sel, [_splat(sp0 >> 7), _splat(sp0 & 127)])
                for h in range(2):
                    k = iota + 16 * h
                    spk = sp0 + k
                    cur = plsc.load_gather(sel, [spk >> 7, spk & 127])
                    new = jnp.where(k < cnt, cur, fsv)
                    new = jnp.where(cnt > 0, new, 0)
                    plsc.store_scatter(sel, [spk >> 7, spk & 127], new)
                plsc.store_scatter(cntv, [_splat(0), _splat(0) + i],
                                   _splat(0) + cnt, mask=iota == 0)
                return bc

            lax.fori_loop(0, G, select_body, 0)

            ecps = []
            for j in range(NCH_SEL):
                cp = pltpu.make_async_copy(feat_h.at[sel.at[j]],
                                           gfeat.at[j], seme)
                cp.start()
                ecps.append(cp)
                cp = pltpu.make_async_copy(xyzp_h.at[sel.at[j]],
                                           gxyz.at[j], seme)
                cp.start()
                ecps.append(cp)
            for cp in ecps:
                cp.wait()

            def out_body(i, bc):
                lq = g * G + i
                xq = _elem(qbuf, lq * 3)
                yq = _elem(qbuf, lq * 3 + 1)
                zq = _elem(qbuf, lq * 3 + 2)
                cz = plsc.load_gather(cntv, [_splat(0), _splat(0) + i]) > 0
                isp = _splat(0) + i
                qs = (xq, yq, zq)
                for h in range(2):
                    sp = _splat(i * NSAMPLE + 16 * h) + iota
                    sr, sc = sp >> 7, sp & 127
                    k = _splat(16 * h) + iota
                    for c in range(3 + CFEAT):
                        if c < 3:
                            v = plsc.load_gather(
                                gxyz, [sr, sc, _splat(c)]) - qs[c]
                        else:
                            v = plsc.load_gather(
                                gfeat, [sr, sc, _splat(c - 3)])
                        v = jnp.where(cz, v, 0.0)
                        plsc.store_scatter(outb, [isp, _splat(c), k], v)
                return bc

            lax.fori_loop(0, G, out_body, 0)

            pltpu.sync_copy(outb, out_h.at[pl.ds(qbase + g * G, G)])
        return carry

    lax.fori_loop(0, NGRP // 2, pair_body, 0)


def _impl(xyz, features, query_rv_xyz, query_rv_coords, rv_map):
    xyzp = jnp.concatenate(
        [xyz, jnp.zeros((xyz.shape[0], 5), jnp.float32)], axis=1)
    rvf = rv_map.reshape(-1)
    qv = query_rv_xyz.reshape(M * 3 // CH, CH)
    cv = query_rv_coords.reshape(M * 3 // CH, CH)

    mesh = plsc.VectorSubcoreMesh(core_axis_name="c", subcore_axis_name="s",
                                  num_cores=NCORES, num_subcores=NSUBC)
    run = pl.kernel(
        _sc_body,
        out_type=jax.ShapeDtypeStruct((M, 3 + CFEAT, NSAMPLE),
                                      jnp.float32),
        mesh=mesh,
        compiler_params=pltpu.CompilerParams(use_tc_tiling_on_sc=False,
                                             needs_layout_passes=False),
        scratch_types=[
            pltpu.VMEM((QPW * 3 // CH, CH), jnp.float32),
            pltpu.VMEM((QPW * 3 // CH, CH), jnp.int32),
            pltpu.VMEM((NCH_CAND, CH), jnp.int32),
            pltpu.VMEM((NCH_CAND, CH), jnp.int32),
            pltpu.VMEM((NCH_CAND, CH), jnp.int32),
            pltpu.VMEM((NCH_CAND, CH), jnp.int32),
            pltpu.VMEM((NCH_CAND, CH, 8), jnp.float32),
            pltpu.VMEM((NCH_CAND, CH, 8), jnp.float32),
            pltpu.VMEM((NCH_SEL, CH), jnp.int32),
            pltpu.VMEM((NCH_SEL, CH, CFEAT), jnp.float32),
            pltpu.VMEM((NCH_SEL, CH, 8), jnp.float32),
            pltpu.VMEM((1, CH), jnp.int32),
            pltpu.VMEM((G, 3 + CFEAT, NSAMPLE), jnp.float32),
            pltpu.SemaphoreType.DMA,
            pltpu.SemaphoreType.DMA,
            pltpu.SemaphoreType.DMA,
        ],
    )
    return run(qv, cv, rvf, xyzp, features)


_impl.__name__ = "kernel"
_JIT = None


def kernel(xyz, features, query_rv_xyz, query_rv_coords, rv_map):
    global _JIT
    if _JIT is None:
        _JIT = jax.jit(_impl)
    return _JIT(xyz, features, query_rv_xyz, query_rv_coords, rv_map)


# speedup vs baseline: 1.2198x; 1.0871x over previous
import functools

import jax
import jax.numpy as jnp
from jax import lax
from jax.experimental import pallas as pl
from jax.experimental.pallas import tpu as pltpu
from jax.experimental.pallas import tpu_sc as plsc

RADIUS2 = 4.0
NSAMPLE = 32
NCAND = 180
NCP = 192
M = 16384
CFEAT = 16
RV_H, RV_W, PPP = 64, 2048, 4

NCORES, NSUBC = 2, 16
NW = NCORES * NSUBC
QPW = M // NW
G = 16
NGRP = QPW // G
CH = 128
NCH_CAND = (G * NCP) // CH
NCH_SEL = (G * NSAMPLE) // CH
OW = (3 + CFEAT) * NSAMPLE


def _splat(x, dtype=jnp.int32):
    return jnp.full((16,), x, dtype=dtype)


def _vgather(v, idx):
    return v.at[idx].get(mode="promise_in_bounds")


def _elem(buf, pos):
    return plsc.load_gather(buf, [_splat(pos >> 7), _splat(pos & 127)])


def _sc_body(q_h, c_h, rvf_h, xyzp_h, feat_h, out_h,
             qbuf, cbuf, eidx0, eidx1, cand0, cand1, cxyz0, cxyz1,
             sel, gfeat, gxyz, cntv, outb, semb, semc, seme):
    wid = lax.axis_index("s") * NCORES + lax.axis_index("c")
    qbase = wid * QPW
    qrow = wid * (QPW * 3 // CH)
    iota = jnp.arange(16, dtype=jnp.int32)

    pltpu.sync_copy(q_h.at[pl.ds(qrow, QPW * 3 // CH)], qbuf)
    pltpu.sync_copy(c_h.at[pl.ds(qrow, QPW * 3 // CH)], cbuf)

    eidxs = (eidx0, eidx1)
    cands = (cand0, cand1)
    cxyzs = (cxyz0, cxyz1)

    def build_group(g, eidx):
        def build_body(i, bc):
            lq = g * G + i
            rsp = _elem(cbuf, lq * 3 + 1) & jnp.int32(RV_H - 1)
            csp = _elem(cbuf, lq * 3 + 2) & jnp.int32(RV_W - 1)
            cells = []
            for jj in range(3):
                u = iota + 16 * jj
                oh = u // 9 - 2
                ow = 2 * (u % 9) - 8
                rr = jnp.clip(rsp + oh, 0, RV_H - 1)
                cc = (csp + ow) & jnp.int32(RV_W - 1)
                cells.append(rr * RV_W + cc)
            for jj2 in range(12):
                lidx = iota // 4 + 4 * (jj2 % 4)
                cv = _vgather(cells[jj2 // 4], lidx)
                ev = cv * PPP + (iota & 3)
                p = _splat(i * NCP + 16 * jj2) + iota
                plsc.store_scatter(eidx, [p >> 7, p & 127], ev)
            return bc

        lax.fori_loop(0, G, build_body, 0)

    def fire_rv(eidx, cand):
        for j in range(NCH_CAND):
            pltpu.make_async_copy(rvf_h.at[eidx.at[j]], cand.at[j],
                                  semb).start()

    def drain_rv(eidx, cand):
        for j in range(NCH_CAND):
            pltpu.make_async_copy(rvf_h.at[eidx.at[j]], cand.at[j],
                                  semb).wait()

    def fire_xyz(cand, cxyz):
        for j in range(NCH_CAND):
            pltpu.make_async_copy(xyzp_h.at[cand.at[j]], cxyz.at[j],
                                  semc).start()

    def drain_xyz(cand, cxyz):
        for j in range(NCH_CAND):
            pltpu.make_async_copy(xyzp_h.at[cand.at[j]], cxyz.at[j],
                                  semc).wait()

    build_group(0, eidx0)
    fire_rv(eidx0, cand0)
    drain_rv(eidx0, cand0)
    fire_xyz(cand0, cxyz0)
    build_group(1, eidx1)
    fire_rv(eidx1, cand1)

    def pair_body(t, carry):
        for par in range(2):
            g = 2 * t + par
            eidx, cand, cxyz = eidxs[par], cands[par], cxyzs[par]
            neidx, ncand, ncxyz = (eidxs[1 - par], cands[1 - par],
                                   cxyzs[1 - par])

            drain_xyz(cand, cxyz)

            @pl.when(g < NGRP - 1)
            def _():
                drain_rv(neidx, ncand)
                fire_xyz(ncand, ncxyz)

            @pl.when(g < NGRP - 2)
            def _():
                build_group(g + 2, eidx)

            def select_body(i, bc):
                lq = g * G + i
                xq = _elem(qbuf, lq * 3)
                yq = _elem(qbuf, lq * 3 + 1)
                zq = _elem(qbuf, lq * 3 + 2)
                z16 = _splat(0)
                cnt = jnp.int32(0)
                for jj in range(12):
                    p = _splat(i * NCP + 16 * jj) + iota
                    pr, pc = p >> 7, p & 127
                    cd = plsc.load_gather(cand, [pr, pc])
                    x = plsc.load_gather(cxyz, [pr, pc, z16])
                    y = plsc.load_gather(cxyz, [pr, pc, z16 + 1])
                    z = plsc.load_gather(cxyz, [pr, pc, z16 + 2])
                    dx, dy, dz = x - xq, y - yq, z - zq
                    d2 = dx * dx + dy * dy + dz * dz
                    val = d2 < RADIUS2
                    if jj == 11:
                        val = val & (iota < (NCAND - 16 * 11))
                    vi = val.astype(jnp.int32)
                    pref = plsc.cumsum(vi)
                    rank = cnt + pref - 1
                    m = val & (rank < NSAMPLE)
                    sp = _splat(i * NSAMPLE) + rank
                    plsc.store_scatter(sel, [sp >> 7, sp & 127], cd, mask=m)
                    cnt = cnt + jnp.sum(vi)
                sp0 = i * NSAMPLE
                fsv = plsc.load_gather(
                    sel, [_splat(sp0 >> 7), _splat(sp0 & 127)])
                for h in range(2):
                    k = iota + 16 * h
                    spk = sp0 + k
                    cur = plsc.load_gather(sel, [spk >> 7, spk & 127])
                    new = jnp.where(k < cnt, cur, fsv)
                    new = jnp.where(cnt > 0, new, 0)
                    plsc.store_scatter(sel, [spk >> 7, spk & 127], new)
                plsc.store_scatter(cntv, [_splat(0), _splat(0) + i],
                                   _splat(0) + cnt, mask=iota == 0)
                return bc

            lax.fori_loop(0, G, select_body, 0)

            @pl.when(g < NGRP - 2)
            def _():
                fire_rv(eidx, cand)

            ecps = []
            for j in range(NCH_SEL):
                cp = pltpu.make_async_copy(feat_h.at[sel.at[j]],
                                           gfeat.at[j], seme)
                cp.start()
                ecps.append(cp)
                cp = pltpu.make_async_copy(xyzp_h.at[sel.at[j]],
                                           gxyz.at[j], seme)
                cp.start()
                ecps.append(cp)
            for cp in ecps:
                cp.wait()

            def out_body(i, bc):
                lq = g * G + i
                xq = _elem(qbuf, lq * 3)
                yq = _elem(qbuf, lq * 3 + 1)
                zq = _elem(qbuf, lq * 3 + 2)
                cz = plsc.load_gather(cntv, [_splat(0), _splat(0) + i]) > 0
                isp = _splat(0) + i
                qs = (xq, yq, zq)
                for h in range(2):
                    sp = _splat(i * NSAMPLE + 16 * h) + iota
                    sr, sc = sp >> 7, sp & 127
                    k = _splat(16 * h) + iota
                    for c in range(3 + CFEAT):
                        if c < 3:
                            v = plsc.load_gather(
                                gxyz, [sr, sc, _splat(c)]) - qs[c]
                        else:
                            v = plsc.load_gather(
                                gfeat, [sr, sc, _splat(c - 3)])
                        v = jnp.where(cz, v, 0.0)
                        plsc.store_scatter(outb, [isp, _splat(c), k], v)
                return bc

            lax.fori_loop(0, G, out_body, 0)

            pltpu.sync_copy(outb, out_h.at[pl.ds(qbase + g * G, G)])
        return carry

    lax.fori_loop(0, NGRP // 2, pair_body, 0)


def _impl(xyz, features, query_rv_xyz, query_rv_coords, rv_map):
    xyzp = jnp.concatenate(
        [xyz, jnp.zeros((xyz.shape[0], 5), jnp.float32)], axis=1)
    rvf = rv_map.reshape(-1)
    qv = query_rv_xyz.reshape(M * 3 // CH, CH)
    cv = query_rv_coords.reshape(M * 3 // CH, CH)

    mesh = plsc.VectorSubcoreMesh(core_axis_name="c", subcore_axis_name="s",
                                  num_cores=NCORES, num_subcores=NSUBC)
    run = pl.kernel(
        _sc_body,
        out_type=jax.ShapeDtypeStruct((M, 3 + CFEAT, NSAMPLE),
                                      jnp.float32),
        mesh=mesh,
        compiler_params=pltpu.CompilerParams(use_tc_tiling_on_sc=False,
                                             needs_layout_passes=False),
        scratch_types=[
            pltpu.VMEM((QPW * 3 // CH, CH), jnp.float32),
            pltpu.VMEM((QPW * 3 // CH, CH), jnp.int32),
            pltpu.VMEM((NCH_CAND, CH), jnp.int32),
            pltpu.VMEM((NCH_CAND, CH), jnp.int32),
            pltpu.VMEM((NCH_CAND, CH), jnp.int32),
            pltpu.VMEM((NCH_CAND, CH), jnp.int32),
            pltpu.VMEM((NCH_CAND, CH, 8), jnp.float32),
            pltpu.VMEM((NCH_CAND, CH, 8), jnp.float32),
            pltpu.VMEM((NCH_SEL, CH), jnp.int32),
            pltpu.VMEM((NCH_SEL, CH, CFEAT), jnp.float32),
            pltpu.VMEM((NCH_SEL, CH, 8), jnp.float32),
            pltpu.VMEM((1, CH), jnp.int32),
            pltpu.VMEM((G, 3 + CFEAT, NSAMPLE), jnp.float32),
            pltpu.SemaphoreType.DMA,
            pltpu.SemaphoreType.DMA,
            pltpu.SemaphoreType.DMA,
        ],
    )
    return run(qv, cv, rvf, xyzp, features)


_impl.__name__ = "kernel"
_JIT = None


def kernel(xyz, features, query_rv_xyz, query_rv_coords, rv_map):
    global _JIT
    if _JIT is None:
        _JIT = jax.jit(_impl)
    return _JIT(xyz, features, query_rv_xyz, query_rv_coords, rv_map)


# speedup vs baseline: 1.2574x; 1.0308x over previous
import functools

import jax
import jax.numpy as jnp
from jax import lax
from jax.experimental import pallas as pl
from jax.experimental.pallas import tpu as pltpu
from jax.experimental.pallas import tpu_sc as plsc

RADIUS2 = 4.0
NSAMPLE = 32
NCAND = 180
NCP = 192
M = 16384
CFEAT = 16
RV_H, RV_W, PPP = 64, 2048, 4

NCORES, NSUBC = 2, 16
NW = NCORES * NSUBC
QPW = M // NW
G = 16
NGRP = QPW // G
CH = 128
NCH_CAND = (G * NCP) // CH
NCH_SEL = (G * NSAMPLE) // CH
OW = (3 + CFEAT) * NSAMPLE


def _splat(x, dtype=jnp.int32):
    return jnp.full((16,), x, dtype=dtype)


def _vgather(v, idx):
    return v.at[idx].get(mode="promise_in_bounds")


def _elem(buf, pos):
    return plsc.load_gather(buf, [_splat(pos >> 7), _splat(pos & 127)])


def _sc_body(q_h, c_h, rvf_h, xyzp_h, feat_h, out_h,
             qbuf, cbuf, eidx0, eidx1, cand0, cand1, cxyz0, cxyz1,
             sel, gfeat, gxyz, cntv, outb0, outb1, semb, semc, seme, semg):
    wid = lax.axis_index("s") * NCORES + lax.axis_index("c")
    qbase = wid * QPW
    qrow = wid * (QPW * 3 // CH)
    iota = jnp.arange(16, dtype=jnp.int32)

    pltpu.sync_copy(q_h.at[pl.ds(qrow, QPW * 3 // CH)], qbuf)
    pltpu.sync_copy(c_h.at[pl.ds(qrow, QPW * 3 // CH)], cbuf)

    eidxs = (eidx0, eidx1)
    cands = (cand0, cand1)
    cxyzs = (cxyz0, cxyz1)
    outbs = (outb0, outb1)

    def build_group(g, eidx):
        def build_body(i, bc):
            lq = g * G + i
            rsp = _elem(cbuf, lq * 3 + 1) & jnp.int32(RV_H - 1)
            csp = _elem(cbuf, lq * 3 + 2) & jnp.int32(RV_W - 1)
            cells = []
            for jj in range(3):
                u = iota + 16 * jj
                oh = u // 9 - 2
                ow = 2 * (u % 9) - 8
                rr = jnp.clip(rsp + oh, 0, RV_H - 1)
                cc = (csp + ow) & jnp.int32(RV_W - 1)
                cells.append(rr * RV_W + cc)
            for jj2 in range(12):
                lidx = iota // 4 + 4 * (jj2 % 4)
                cv = _vgather(cells[jj2 // 4], lidx)
                ev = cv * PPP + (iota & 3)
                p = _splat(i * NCP + 16 * jj2) + iota
                plsc.store_scatter(eidx, [p >> 7, p & 127], ev)
            return bc

        lax.fori_loop(0, G, build_body, 0)

    def fire_rv(eidx, cand):
        for j in range(NCH_CAND):
            pltpu.make_async_copy(rvf_h.at[eidx.at[j]], cand.at[j],
                                  semb).start()

    def drain_rv(eidx, cand):
        for j in range(NCH_CAND):
            pltpu.make_async_copy(rvf_h.at[eidx.at[j]], cand.at[j],
                                  semb).wait()

    def fire_xyz(cand, cxyz):
        for j in range(NCH_CAND):
            pltpu.make_async_copy(xyzp_h.at[cand.at[j]], cxyz.at[j],
                                  semc).start()

    def drain_xyz(cand, cxyz):
        for j in range(NCH_CAND):
            pltpu.make_async_copy(xyzp_h.at[cand.at[j]], cxyz.at[j],
                                  semc).wait()

    build_group(0, eidx0)
    fire_rv(eidx0, cand0)
    drain_rv(eidx0, cand0)
    fire_xyz(cand0, cxyz0)
    build_group(1, eidx1)
    fire_rv(eidx1, cand1)

    def pair_body(t, carry):
        for par in range(2):
            g = 2 * t + par
            eidx, cand, cxyz = eidxs[par], cands[par], cxyzs[par]
            neidx, ncand, ncxyz = (eidxs[1 - par], cands[1 - par],
                                   cxyzs[1 - par])

            drain_xyz(cand, cxyz)

            @pl.when(g < NGRP - 1)
            def _():
                drain_rv(neidx, ncand)
                fire_xyz(ncand, ncxyz)

            @pl.when(g < NGRP - 2)
            def _():
                build_group(g + 2, eidx)

            def select_body(i, bc):
                lq = g * G + i
                xq = _elem(qbuf, lq * 3)
                yq = _elem(qbuf, lq * 3 + 1)
                zq = _elem(qbuf, lq * 3 + 2)
                z16 = _splat(0)
                cnt = jnp.int32(0)
                for jj in range(12):
                    p = _splat(i * NCP + 16 * jj) + iota
                    pr, pc = p >> 7, p & 127
                    cd = plsc.load_gather(cand, [pr, pc])
                    x = plsc.load_gather(cxyz, [pr, pc, z16])
                    y = plsc.load_gather(cxyz, [pr, pc, z16 + 1])
                    z = plsc.load_gather(cxyz, [pr, pc, z16 + 2])
                    dx, dy, dz = x - xq, y - yq, z - zq
                    d2 = dx * dx + dy * dy + dz * dz
                    val = d2 < RADIUS2
                    if jj == 11:
                        val = val & (iota < (NCAND - 16 * 11))
                    vi = val.astype(jnp.int32)
                    pref = plsc.cumsum(vi)
                    rank = cnt + pref - 1
                    m = val & (rank < NSAMPLE)
                    sp = _splat(i * NSAMPLE) + rank
                    plsc.store_scatter(sel, [sp >> 7, sp & 127], cd, mask=m)
                    cnt = cnt + jnp.sum(vi)
                sp0 = i * NSAMPLE
                fsv = plsc.load_gather(
                    sel, [_splat(sp0 >> 7), _splat(sp0 & 127)])
                for h in range(2):
                    k = iota + 16 * h
                    spk = sp0 + k
                    cur = plsc.load_gather(sel, [spk >> 7, spk & 127])
                    new = jnp.where(k < cnt, cur, fsv)
                    new = jnp.where(cnt > 0, new, 0)
                    plsc.store_scatter(sel, [spk >> 7, spk & 127], new)
                plsc.store_scatter(cntv, [_splat(0), _splat(0) + i],
                                   _splat(0) + cnt, mask=iota == 0)
                @pl.when((i & 3) == 3)
                def _():
                    j = i >> 2
                    pltpu.make_async_copy(feat_h.at[sel.at[j]],
                                          gfeat.at[j], seme).start()
                    pltpu.make_async_copy(xyzp_h.at[sel.at[j]],
                                          gxyz.at[j], seme).start()
                return bc

            lax.fori_loop(0, G, select_body, 0)

            @pl.when(g < NGRP - 2)
            def _():
                fire_rv(eidx, cand)

            for j in range(NCH_SEL):
                pltpu.make_async_copy(feat_h.at[sel.at[j]], gfeat.at[j],
                                      seme).wait()
                pltpu.make_async_copy(xyzp_h.at[sel.at[j]], gxyz.at[j],
                                      seme).wait()

            outb = outbs[par]
            @pl.when(g >= 2)
            def _():
                pltpu.make_async_copy(
                    outb, out_h.at[pl.ds(qbase + (g - 2) * G, G)],
                    semg).wait()

            def out_body(i, bc):
                lq = g * G + i
                xq = _elem(qbuf, lq * 3)
                yq = _elem(qbuf, lq * 3 + 1)
                zq = _elem(qbuf, lq * 3 + 2)
                cz = plsc.load_gather(cntv, [_splat(0), _splat(0) + i]) > 0
                isp = _splat(0) + i
                qs = (xq, yq, zq)
                for h in range(2):
                    sp = _splat(i * NSAMPLE + 16 * h) + iota
                    sr, sc = sp >> 7, sp & 127
                    k = _splat(16 * h) + iota
                    for c in range(3 + CFEAT):
                        if c < 3:
                            v = plsc.load_gather(
                                gxyz, [sr, sc, _splat(c)]) - qs[c]
                        else:
                            v = plsc.load_gather(
                                gfeat, [sr, sc, _splat(c - 3)])
                        v = jnp.where(cz, v, 0.0)
                        plsc.store_scatter(outb, [isp, _splat(c), k], v)
                return bc

            lax.fori_loop(0, G, out_body, 0)

            pltpu.make_async_copy(
                outb, out_h.at[pl.ds(qbase + g * G, G)], semg).start()
        return carry

    lax.fori_loop(0, NGRP // 2, pair_body, 0)

    for gl in (NGRP - 2, NGRP - 1):
        pltpu.make_async_copy(
            outbs[gl % 2], out_h.at[pl.ds(qbase + gl * G, G)],
            semg).wait()


def _impl(xyz, features, query_rv_xyz, query_rv_coords, rv_map):
    xyzp = jnp.concatenate(
        [xyz, jnp.zeros((xyz.shape[0], 5), jnp.float32)], axis=1)
    rvf = rv_map.reshape(-1)
    qv = query_rv_xyz.reshape(M * 3 // CH, CH)
    cv = query_rv_coords.reshape(M * 3 // CH, CH)

    mesh = plsc.VectorSubcoreMesh(core_axis_name="c", subcore_axis_name="s",
                                  num_cores=NCORES, num_subcores=NSUBC)
    run = pl.kernel(
        _sc_body,
        out_type=jax.ShapeDtypeStruct((M, 3 + CFEAT, NSAMPLE),
                                      jnp.float32),
        mesh=mesh,
        compiler_params=pltpu.CompilerParams(use_tc_tiling_on_sc=False,
                                             needs_layout_passes=False),
        scratch_types=[
            pltpu.VMEM((QPW * 3 // CH, CH), jnp.float32),
            pltpu.VMEM((QPW * 3 // CH, CH), jnp.int32),
            pltpu.VMEM((NCH_CAND, CH), jnp.int32),
            pltpu.VMEM((NCH_CAND, CH), jnp.int32),
            pltpu.VMEM((NCH_CAND, CH), jnp.int32),
            pltpu.VMEM((NCH_CAND, CH), jnp.int32),
            pltpu.VMEM((NCH_CAND, CH, 8), jnp.float32),
            pltpu.VMEM((NCH_CAND, CH, 8), jnp.float32),
            pltpu.VMEM((NCH_SEL, CH), jnp.int32),
            pltpu.VMEM((NCH_SEL, CH, CFEAT), jnp.float32),
            pltpu.VMEM((NCH_SEL, CH, 8), jnp.float32),
            pltpu.VMEM((1, CH), jnp.int32),
            pltpu.VMEM((G, 3 + CFEAT, NSAMPLE), jnp.float32),
            pltpu.VMEM((G, 3 + CFEAT, NSAMPLE), jnp.float32),
            pltpu.SemaphoreType.DMA,
            pltpu.SemaphoreType.DMA,
            pltpu.SemaphoreType.DMA,
            pltpu.SemaphoreType.DMA,
        ],
    )
    return run(qv, cv, rvf, xyzp, features)


_impl.__name__ = "kernel"
_JIT = None


def kernel(xyz, features, query_rv_xyz, query_rv_coords, rv_map):
    global _JIT
    if _JIT is None:
        _JIT = jax.jit(_impl)
    return _JIT(xyz, features, query_rv_xyz, query_rv_coords, rv_map)
